# Initial kernel scaffold; baseline (speedup 1.0000x reference)
import jax
import jax.numpy as jnp
from jax.experimental import pallas as pl


def kernel(x, params):
    raise NotImplementedError("write your pallas kernel here")


# speedup vs baseline: 1.6566x; 1.6566x over previous
import functools

import jax
import jax.numpy as jnp
from jax.experimental import pallas as pl
from jax.experimental.pallas import tpu as pltpu


def _fps_body(xs_ref, ys_ref, zs_ref, idx_ref, cx_ref, cy_ref, cz_ref, *, npoint):
    B, N = xs_ref.shape
    xs = xs_ref[...]
    ys = ys_ref[...]
    zs = zs_ref[...]
    iota = jax.lax.broadcasted_iota(jnp.int32, (B, N), 1)
    eye = (jax.lax.broadcasted_iota(jnp.int32, (B, B), 0)
           == jax.lax.broadcasted_iota(jnp.int32, (B, B), 1))

    def to_row(col):
        return jnp.sum(jnp.where(eye, col, jnp.zeros_like(col)), axis=0,
                       keepdims=True)

    def step(t, carry):
        dist, far = carry
        mask = iota == far
        cx = jnp.sum(jnp.where(mask, xs, 0.0), axis=1, keepdims=True)
        cy = jnp.sum(jnp.where(mask, ys, 0.0), axis=1, keepdims=True)
        cz = jnp.sum(jnp.where(mask, zs, 0.0), axis=1, keepdims=True)
        idx_ref[pl.ds(t, 1), :, :] = to_row(far)[None]
        cx_ref[pl.ds(t, 1), :, :] = to_row(cx)[None]
        cy_ref[pl.ds(t, 1), :, :] = to_row(cy)[None]
        cz_ref[pl.ds(t, 1), :, :] = to_row(cz)[None]
        d = (xs - cx) ** 2 + (ys - cy) ** 2 + (zs - cz) ** 2
        dist = jnp.minimum(dist, d)
        m = jnp.max(dist, axis=1, keepdims=True)
        cand = jnp.where(dist == m, iota, jnp.int32(N))
        nxt = jnp.min(cand, axis=1, keepdims=True)
        return dist, nxt

    d0 = jnp.full((B, N), 1e10, dtype=jnp.float32)
    f0 = jnp.zeros((B, 1), dtype=jnp.int32)
    jax.lax.fori_loop(0, npoint, step, (d0, f0))


def _fps(xyz, npoint):
    B, N, _ = xyz.shape
    xs = xyz[:, :, 0]
    ys = xyz[:, :, 1]
    zs = xyz[:, :, 2]
    out_shapes = (
        jax.ShapeDtypeStruct((npoint, 1, B), jnp.int32),
        jax.ShapeDtypeStruct((npoint, 1, B), jnp.float32),
        jax.ShapeDtypeStruct((npoint, 1, B), jnp.float32),
        jax.ShapeDtypeStruct((npoint, 1, B), jnp.float32),
    )
    idx, cx, cy, cz = pl.pallas_call(
        functools.partial(_fps_body, npoint=npoint),
        out_shape=out_shapes,
    )(xs, ys, zs)
    idx = idx[:, 0, :].T
    new_xyz = jnp.stack([cx[:, 0, :].T, cy[:, 0, :].T, cz[:, 0, :].T], axis=-1)
    return idx, new_xyz


def _index_points(points, idx):
    if idx.ndim == 2:
        return jnp.take_along_axis(points, idx[:, :, None], axis=1)
    B, S, K = idx.shape
    flat = idx.reshape(B, S * K)
    out = jnp.take_along_axis(points, flat[:, :, None], axis=1)
    return out.reshape(B, S, K, points.shape[-1])


def _knn(query, base, k):
    d = (jnp.sum(query ** 2, -1, keepdims=True)
         + jnp.sum(base ** 2, -1)[:, None, :]
         - 2.0 * jnp.einsum('bsd,bnd->bsn', query, base))
    _, idx = jax.lax.top_k(-d, k)
    return idx


def _sa_layer(xyz, points, npoint, nsample, layers, group_all):
    if group_all:
        new_xyz = xyz[:, :1, :]
        grouped = xyz[:, None, :, :]
        if points is not None:
            grouped = jnp.concatenate([grouped, points[:, None, :, :]], axis=-1)
    else:
        fidx, new_xyz = _fps(xyz, npoint)
        idx = _knn(new_xyz, xyz, nsample)
        grouped_xyz = _index_points(xyz, idx) - new_xyz[:, :, None, :]
        if points is not None:
            grouped = jnp.concatenate([grouped_xyz, _index_points(points, idx)], axis=-1)
        else:
            grouped = grouped_xyz
    h = grouped
    for (W, b, g, be) in layers:
        h = jnp.einsum('bskc,oc->bsko', h, W) + b
        mean = jnp.mean(h, axis=(0, 1, 2))
        var = jnp.var(h, axis=(0, 1, 2))
        h = (h - mean) / jnp.sqrt(var + 1e-5) * g + be
        h = jax.nn.relu(h)
    return new_xyz, jnp.max(h, axis=2)


def kernel(x, params):
    xyz, points = x, None
    xyz, points = _sa_layer(xyz, points, 512, 32, params['sa1'], False)
    xyz, points = _sa_layer(xyz, points, 128, 32, params['sa2'], False)
    xyz, points = _sa_layer(xyz, points, 32, 32, params['sa3'], False)
    xyz, points = _sa_layer(xyz, points, None, None, params['sa4'], True)
    return points[:, 0, :]


# speedup vs baseline: 4.1896x; 2.5289x over previous
import functools

import jax
import jax.numpy as jnp
from jax.experimental import pallas as pl
from jax.experimental.pallas import tpu as pltpu


def _fps_body(xs_ref, ys_ref, zs_ref, idx_ref, cx_ref, cy_ref, cz_ref, *, npoint):
    B, N = xs_ref.shape
    xs = xs_ref[...]
    ys = ys_ref[...]
    zs = zs_ref[...]
    iota = jax.lax.broadcasted_iota(jnp.int32, (B, N), 1)
    eye = (jax.lax.broadcasted_iota(jnp.int32, (B, B), 0)
           == jax.lax.broadcasted_iota(jnp.int32, (B, B), 1))

    def to_row(col):
        return jnp.sum(jnp.where(eye, col, jnp.zeros_like(col)), axis=0,
                       keepdims=True)

    def step(t, carry):
        dist, far = carry
        mask = iota == far
        cx = jnp.sum(jnp.where(mask, xs, 0.0), axis=1, keepdims=True)
        cy = jnp.sum(jnp.where(mask, ys, 0.0), axis=1, keepdims=True)
        cz = jnp.sum(jnp.where(mask, zs, 0.0), axis=1, keepdims=True)
        idx_ref[pl.ds(t, 1), :, :] = to_row(far)[None]
        cx_ref[pl.ds(t, 1), :, :] = to_row(cx)[None]
        cy_ref[pl.ds(t, 1), :, :] = to_row(cy)[None]
        cz_ref[pl.ds(t, 1), :, :] = to_row(cz)[None]
        d = (xs - cx) ** 2 + (ys - cy) ** 2 + (zs - cz) ** 2
        dist = jnp.minimum(dist, d)
        m = jnp.max(dist, axis=1, keepdims=True)
        cand = jnp.where(dist == m, iota, jnp.int32(N))
        nxt = jnp.min(cand, axis=1, keepdims=True)
        return dist, nxt

    d0 = jnp.full((B, N), 1e10, dtype=jnp.float32)
    f0 = jnp.zeros((B, 1), dtype=jnp.int32)
    jax.lax.fori_loop(0, npoint, step, (d0, f0))


def _fps(xyz, npoint):
    B, N, _ = xyz.shape
    xs = xyz[:, :, 0]
    ys = xyz[:, :, 1]
    zs = xyz[:, :, 2]
    out_shapes = (
        jax.ShapeDtypeStruct((npoint, 1, B), jnp.int32),
        jax.ShapeDtypeStruct((npoint, 1, B), jnp.float32),
        jax.ShapeDtypeStruct((npoint, 1, B), jnp.float32),
        jax.ShapeDtypeStruct((npoint, 1, B), jnp.float32),
    )
    idx, cx, cy, cz = pl.pallas_call(
        functools.partial(_fps_body, npoint=npoint),
        out_shape=out_shapes,
    )(xs, ys, zs)
    idx = idx[:, 0, :].T
    new_xyz = jnp.stack([cx[:, 0, :].T, cy[:, 0, :].T, cz[:, 0, :].T], axis=-1)
    return idx, new_xyz


def _index_points(points, idx):
    if idx.ndim == 2:
        return jnp.take_along_axis(points, idx[:, :, None], axis=1)
    B, S, K = idx.shape
    flat = idx.reshape(B, S * K)
    out = jnp.take_along_axis(points, flat[:, :, None], axis=1)
    return out.reshape(B, S, K, points.shape[-1])


def _knn_body(q_ref, bx_ref, by_ref, bz_ref, idx_ref, *, K):
    q = q_ref[0]
    S = q.shape[0]
    N = bx_ref.shape[-1]
    qx, qy, qz = q[:, 0:1], q[:, 1:2], q[:, 2:3]
    bx, by, bz = bx_ref[0], by_ref[0], bz_ref[0]
    sq = qx * qx + qy * qy + qz * qz
    sb = bx * bx + by * by + bz * bz
    def r(v):
        return v.astype(jnp.bfloat16).astype(jnp.float32)
    qb = r(qx) * r(bx) + r(qy) * r(by) + r(qz) * r(bz)
    D = sq + sb - 2.0 * qb
    iota = jax.lax.broadcasted_iota(jnp.int32, (S, N), 1)
    kiota = jax.lax.broadcasted_iota(jnp.int32, (S, K), 1)
    acc = jnp.zeros((S, K), jnp.int32)
    INF = jnp.float32(3.0e38)
    for k in range(K):
        m = jnp.min(D, axis=1, keepdims=True)
        cand = jnp.where(D == m, iota, jnp.int32(N))
        j = jnp.min(cand, axis=1, keepdims=True)
        acc = jnp.where(kiota == k, j, acc)
        D = jnp.where(iota == j, INF, D)
    idx_ref[0] = acc


def _knn(new_xyz, bx, by, bz, k):
    B, S, _ = new_xyz.shape
    N = bx.shape[1]
    bx3 = bx[:, None, :]
    by3 = by[:, None, :]
    bz3 = bz[:, None, :]
    return pl.pallas_call(
        functools.partial(_knn_body, K=k),
        grid=(B,),
        in_specs=[
            pl.BlockSpec((1, S, 3), lambda b: (b, 0, 0)),
            pl.BlockSpec((1, 1, N), lambda b: (b, 0, 0)),
            pl.BlockSpec((1, 1, N), lambda b: (b, 0, 0)),
            pl.BlockSpec((1, 1, N), lambda b: (b, 0, 0)),
        ],
        out_specs=pl.BlockSpec((1, S, k), lambda b: (b, 0, 0)),
        out_shape=jax.ShapeDtypeStruct((B, S, k), jnp.int32),
    )(new_xyz, bx3, by3, bz3)


def _sa_layer(xyz, points, npoint, nsample, layers, group_all):
    if group_all:
        new_xyz = xyz[:, :1, :]
        grouped = xyz[:, None, :, :]
        if points is not None:
            grouped = jnp.concatenate([grouped, points[:, None, :, :]], axis=-1)
    else:
        fidx, new_xyz = _fps(xyz, npoint)
        idx = _knn(new_xyz, xyz[:, :, 0], xyz[:, :, 1], xyz[:, :, 2], nsample)
        grouped_xyz = _index_points(xyz, idx) - new_xyz[:, :, None, :]
        if points is not None:
            grouped = jnp.concatenate([grouped_xyz, _index_points(points, idx)], axis=-1)
        else:
            grouped = grouped_xyz
    h = grouped
    for (W, b, g, be) in layers:
        h = jnp.einsum('bskc,oc->bsko', h, W) + b
        mean = jnp.mean(h, axis=(0, 1, 2))
        var = jnp.var(h, axis=(0, 1, 2))
        h = (h - mean) / jnp.sqrt(var + 1e-5) * g + be
        h = jax.nn.relu(h)
    return new_xyz, jnp.max(h, axis=2)


def kernel(x, params):
    xyz, points = x, None
    xyz, points = _sa_layer(xyz, points, 512, 32, params['sa1'], False)
    xyz, points = _sa_layer(xyz, points, 128, 32, params['sa2'], False)
    xyz, points = _sa_layer(xyz, points, 32, 32, params['sa3'], False)
    xyz, points = _sa_layer(xyz, points, None, None, params['sa4'], True)
    return points[:, 0, :]


# speedup vs baseline: 7.5401x; 1.7997x over previous
import functools

import jax
import jax.numpy as jnp
from jax.experimental import pallas as pl
from jax.experimental.pallas import tpu as pltpu


def _fps_body(xs_ref, ys_ref, zs_ref, idx_ref, cx_ref, cy_ref, cz_ref, *, npoint):
    B, N = xs_ref.shape
    xs = xs_ref[...]
    ys = ys_ref[...]
    zs = zs_ref[...]
    iota = jax.lax.broadcasted_iota(jnp.int32, (B, N), 1)
    eye = (jax.lax.broadcasted_iota(jnp.int32, (B, B), 0)
           == jax.lax.broadcasted_iota(jnp.int32, (B, B), 1))

    def to_row(col):
        return jnp.sum(jnp.where(eye, col, jnp.zeros_like(col)), axis=0,
                       keepdims=True)

    def step(t, carry):
        dist, far = carry
        mask = iota == far
        cx = jnp.sum(jnp.where(mask, xs, 0.0), axis=1, keepdims=True)
        cy = jnp.sum(jnp.where(mask, ys, 0.0), axis=1, keepdims=True)
        cz = jnp.sum(jnp.where(mask, zs, 0.0), axis=1, keepdims=True)
        idx_ref[pl.ds(t, 1), :, :] = to_row(far)[None]
        cx_ref[pl.ds(t, 1), :, :] = to_row(cx)[None]
        cy_ref[pl.ds(t, 1), :, :] = to_row(cy)[None]
        cz_ref[pl.ds(t, 1), :, :] = to_row(cz)[None]
        d = (xs - cx) ** 2 + (ys - cy) ** 2 + (zs - cz) ** 2
        dist = jnp.minimum(dist, d)
        m = jnp.max(dist, axis=1, keepdims=True)
        cand = jnp.where(dist == m, iota, jnp.int32(N))
        nxt = jnp.min(cand, axis=1, keepdims=True)
        return dist, nxt

    d0 = jnp.full((B, N), 1e10, dtype=jnp.float32)
    f0 = jnp.zeros((B, 1), dtype=jnp.int32)
    jax.lax.fori_loop(0, npoint, step, (d0, f0))


def _fps(xyz, npoint):
    B, N, _ = xyz.shape
    xs = xyz[:, :, 0]
    ys = xyz[:, :, 1]
    zs = xyz[:, :, 2]
    out_shapes = (
        jax.ShapeDtypeStruct((npoint, 1, B), jnp.int32),
        jax.ShapeDtypeStruct((npoint, 1, B), jnp.float32),
        jax.ShapeDtypeStruct((npoint, 1, B), jnp.float32),
        jax.ShapeDtypeStruct((npoint, 1, B), jnp.float32),
    )
    idx, cx, cy, cz = pl.pallas_call(
        functools.partial(_fps_body, npoint=npoint),
        out_shape=out_shapes,
    )(xs, ys, zs)
    idx = idx[:, 0, :].T
    new_xyz = jnp.stack([cx[:, 0, :].T, cy[:, 0, :].T, cz[:, 0, :].T], axis=-1)
    return idx, new_xyz


_EPS = 1e-5


def _layer_first_body(x_ref, off_ref, wt_ref, b_ref, p_ref, sum_ref, sq_ref):
    X = x_ref[...] - off_ref[...]
    P = jax.lax.dot_general(
        X.astype(jnp.bfloat16), wt_ref[...].astype(jnp.bfloat16),
        (((1,), (0,)), ((), ())), preferred_element_type=jnp.float32)
    P = P + b_ref[...]
    p_ref[...] = P
    s = jnp.sum(P, axis=0, keepdims=True)
    q = jnp.sum(P * P, axis=0, keepdims=True)

    @pl.when(pl.program_id(0) == 0)
    def _():
        sum_ref[...] = s
        sq_ref[...] = q

    @pl.when(pl.program_id(0) > 0)
    def _():
        sum_ref[...] += s
        sq_ref[...] += q


def _norm_relu(X, s_ref, q_ref, g_ref, be_ref, R):
    mu = s_ref[...] / R
    var = q_ref[...] / R - mu * mu
    X = (X - mu) / jnp.sqrt(var + _EPS) * g_ref[...] + be_ref[...]
    return jnp.maximum(X, 0.0)


def _layer_mid_body(x_ref, s_ref, q_ref, g_ref, be_ref, wt_ref, b_ref,
                    p_ref, sum_ref, sq_ref, *, R):
    X = _norm_relu(x_ref[...], s_ref, q_ref, g_ref, be_ref, R)
    P = jax.lax.dot_general(
        X.astype(jnp.bfloat16), wt_ref[...].astype(jnp.bfloat16),
        (((1,), (0,)), ((), ())), preferred_element_type=jnp.float32)
    P = P + b_ref[...]
    p_ref[...] = P
    s = jnp.sum(P, axis=0, keepdims=True)
    q = jnp.sum(P * P, axis=0, keepdims=True)

    @pl.when(pl.program_id(0) == 0)
    def _():
        sum_ref[...] = s
        sq_ref[...] = q

    @pl.when(pl.program_id(0) > 0)
    def _():
        sum_ref[...] += s
        sq_ref[...] += q


def _maxpool_body(x_ref, s_ref, q_ref, g_ref, be_ref, o_ref, *, R, K, C):
    Y = _norm_relu(x_ref[...], s_ref, q_ref, g_ref, be_ref, R)
    acc = Y[:, 0:C]
    for k in range(1, K):
        acc = jnp.maximum(acc, Y[:, k * C:(k + 1) * C])
    o_ref[...] = acc


def _run_layer(X, off, stats, WT, b, tile):
    R, Cin = X.shape
    Cout = WT.shape[1]
    n = R // tile
    outs = (
        jax.ShapeDtypeStruct((R, Cout), jnp.float32),
        jax.ShapeDtypeStruct((1, Cout), jnp.float32),
        jax.ShapeDtypeStruct((1, Cout), jnp.float32),
    )
    row_spec = pl.BlockSpec((tile, Cin), lambda i: (i, 0))
    full = lambda shp: pl.BlockSpec(shp, lambda i: (0, 0))
    out_specs = (
        pl.BlockSpec((tile, Cout), lambda i: (i, 0)),
        full((1, Cout)),
        full((1, Cout)),
    )
    if off is not None:
        return pl.pallas_call(
            _layer_first_body, grid=(n,),
            in_specs=[row_spec, row_spec, full((Cin, Cout)), full((1, Cout))],
            out_specs=out_specs, out_shape=outs,
        )(X, off, WT, b)
    s, q, g, be = stats
    return pl.pallas_call(
        functools.partial(_layer_mid_body, R=float(R)), grid=(n,),
        in_specs=[row_spec, full((1, Cin)), full((1, Cin)), full((1, Cin)),
                  full((1, Cin)), full((Cin, Cout)), full((1, Cout))],
        out_specs=out_specs, out_shape=outs,
    )(X, s, q, g, be, WT, b)


def _run_maxpool(P, stats, B, S, K, C, tile):
    s, q, g, be = stats
    X = P.reshape(B * S, K * C)
    gt = jnp.tile(g, (1, K))
    bt = jnp.tile(be, (1, K))
    st = jnp.tile(s, (1, K))
    qt = jnp.tile(q, (1, K))
    n = (B * S) // tile
    full = lambda shp: pl.BlockSpec(shp, lambda i: (0, 0))
    return pl.pallas_call(
        functools.partial(_maxpool_body, R=float(B * S * K), K=K, C=C),
        grid=(n,),
        in_specs=[pl.BlockSpec((tile, K * C), lambda i: (i, 0)),
                  full((1, K * C)), full((1, K * C)), full((1, K * C)),
                  full((1, K * C))],
        out_specs=pl.BlockSpec((tile, C), lambda i: (i, 0)),
        out_shape=jax.ShapeDtypeStruct((B * S, C), jnp.float32),
    )(X, st, qt, gt, bt)


def _knn_body(q_ref, bx_ref, by_ref, bz_ref, idx_ref, *, K):
    q = q_ref[0]
    S = q.shape[0]
    N = bx_ref.shape[-1]
    qx, qy, qz = q[:, 0:1], q[:, 1:2], q[:, 2:3]
    bx, by, bz = bx_ref[0], by_ref[0], bz_ref[0]
    sq = qx * qx + qy * qy + qz * qz
    sb = bx * bx + by * by + bz * bz
    def r(v):
        return v.astype(jnp.bfloat16).astype(jnp.float32)
    qb = r(qx) * r(bx) + r(qy) * r(by) + r(qz) * r(bz)
    D = sq + sb - 2.0 * qb
    iota = jax.lax.broadcasted_iota(jnp.int32, (S, N), 1)
    kiota = jax.lax.broadcasted_iota(jnp.int32, (S, K), 1)
    acc = jnp.zeros((S, K), jnp.int32)
    INF = jnp.float32(3.0e38)
    for k in range(K):
        m = jnp.min(D, axis=1, keepdims=True)
        cand = jnp.where(D == m, iota, jnp.int32(N))
        j = jnp.min(cand, axis=1, keepdims=True)
        acc = jnp.where(kiota == k, j, acc)
        D = jnp.where(iota == j, INF, D)
    idx_ref[0] = acc


def _knn(new_xyz, bx, by, bz, k):
    B, S, _ = new_xyz.shape
    N = bx.shape[1]
    bx3 = bx[:, None, :]
    by3 = by[:, None, :]
    bz3 = bz[:, None, :]
    return pl.pallas_call(
        functools.partial(_knn_body, K=k),
        grid=(B,),
        in_specs=[
            pl.BlockSpec((1, S, 3), lambda b: (b, 0, 0)),
            pl.BlockSpec((1, 1, N), lambda b: (b, 0, 0)),
            pl.BlockSpec((1, 1, N), lambda b: (b, 0, 0)),
            pl.BlockSpec((1, 1, N), lambda b: (b, 0, 0)),
        ],
        out_specs=pl.BlockSpec((1, S, k), lambda b: (b, 0, 0)),
        out_shape=jax.ShapeDtypeStruct((B, S, k), jnp.int32),
    )(new_xyz, bx3, by3, bz3)


def _gather_rows(table, idx_flat):
    return jnp.take(table, idx_flat, axis=0)


def _sa_stage(xyz, points, npoint, K, layers):
    B, N, _ = xyz.shape
    S = npoint
    (W1, b1, g1, be1), (W2, b2, g2, be2), (W3, b3, g3, be3) = layers
    _, new_xyz = _fps(xyz, npoint)
    idx = _knn(new_xyz, xyz[:, :, 0], xyz[:, :, 1], xyz[:, :, 2], K)
    Cin = 3 + (points.shape[-1] if points is not None else 0)
    Cp = ((Cin + 15) // 16) * 16
    T = xyz if points is None else jnp.concatenate([xyz, points], -1)
    Tp = jnp.pad(T.reshape(B * N, Cin), ((0, 0), (0, Cp - Cin)))
    idx_flat = (idx + (jnp.arange(B, dtype=jnp.int32) * N)[:, None, None]
                ).reshape(-1)
    G = _gather_rows(Tp, idx_flat)
    R = B * S * K
    off = jnp.pad(jnp.repeat(new_xyz, K, axis=1).reshape(R, 3),
                  ((0, 0), (0, Cp - 3)))
    W1Tp = jnp.pad(W1.T, ((0, Cp - Cin), (0, 0)))
    tile = min(R, 8192)
    P1, s1, q1 = _run_layer(G, off, None, W1Tp, b1[None], tile)
    P2, s2, q2 = _run_layer(P1, None, (s1, q1, g1[None], be1[None]),
                            W2.T, b2[None], tile)
    P3, s3, q3 = _run_layer(P2, None, (s2, q2, g2[None], be2[None]),
                            W3.T, b3[None], tile)
    C3 = W3.shape[0]
    tile_d = min(B * S, max(8, (8 * 1024 * 1024) // (K * C3 * 4)))
    out = _run_maxpool(P3, (s3, q3, g3[None], be3[None]), B, S, K, C3, tile_d)
    return new_xyz, out.reshape(B, S, C3)


def _sa4(xyz, points, layers):
    B, S, _ = xyz.shape
    X = jnp.concatenate([xyz, points], -1).reshape(B * S, -1)
    (W1, b1, g1, be1), (W2, b2, g2, be2), (W3, b3, g3, be3) = layers
    R = B * S
    P1, s1, q1 = _run_layer(X, jnp.zeros_like(X), None, W1.T, b1[None], R)
    P2, s2, q2 = _run_layer(P1, None, (s1, q1, g1[None], be1[None]),
                            W2.T, b2[None], R)
    P3, s3, q3 = _run_layer(P2, None, (s2, q2, g2[None], be2[None]),
                            W3.T, b3[None], R)
    return _run_maxpool(P3, (s3, q3, g3[None], be3[None]), B, 1, S, 1024, B)


def kernel(x, params):
    xyz, points = x, None
    xyz, points = _sa_stage(xyz, points, 512, 32, params['sa1'])
    xyz, points = _sa_stage(xyz, points, 128, 32, params['sa2'])
    xyz, points = _sa_stage(xyz, points, 32, 32, params['sa3'])
    return _sa4(xyz, points, params['sa4'])


# speedup vs baseline: 9.3016x; 1.2336x over previous
import functools

import jax
import jax.numpy as jnp
from jax import lax
from jax.experimental import pallas as pl
from jax.experimental.pallas import tpu as pltpu
from jax.experimental.pallas import tpu_sc as plsc


def _fps_body(xs_ref, ys_ref, zs_ref, idx_ref, cx_ref, cy_ref, cz_ref, *, npoint):
    B, N = xs_ref.shape
    xs = xs_ref[...]
    ys = ys_ref[...]
    zs = zs_ref[...]
    iota = jax.lax.broadcasted_iota(jnp.int32, (B, N), 1)
    eye = (jax.lax.broadcasted_iota(jnp.int32, (B, B), 0)
           == jax.lax.broadcasted_iota(jnp.int32, (B, B), 1))

    def to_row(col):
        return jnp.sum(jnp.where(eye, col, jnp.zeros_like(col)), axis=0,
                       keepdims=True)

    def step(t, carry):
        dist, far = carry
        mask = iota == far
        cx = jnp.sum(jnp.where(mask, xs, 0.0), axis=1, keepdims=True)
        cy = jnp.sum(jnp.where(mask, ys, 0.0), axis=1, keepdims=True)
        cz = jnp.sum(jnp.where(mask, zs, 0.0), axis=1, keepdims=True)
        idx_ref[pl.ds(t, 1), :, :] = to_row(far)[None]
        cx_ref[pl.ds(t, 1), :, :] = to_row(cx)[None]
        cy_ref[pl.ds(t, 1), :, :] = to_row(cy)[None]
        cz_ref[pl.ds(t, 1), :, :] = to_row(cz)[None]
        d = (xs - cx) ** 2 + (ys - cy) ** 2 + (zs - cz) ** 2
        dist = jnp.minimum(dist, d)
        m = jnp.max(dist, axis=1, keepdims=True)
        cand = jnp.where(dist == m, iota, jnp.int32(N))
        nxt = jnp.min(cand, axis=1, keepdims=True)
        return dist, nxt

    d0 = jnp.full((B, N), 1e10, dtype=jnp.float32)
    f0 = jnp.zeros((B, 1), dtype=jnp.int32)
    jax.lax.fori_loop(0, npoint, step, (d0, f0))


def _fps(xyz, npoint):
    B, N, _ = xyz.shape
    xs = xyz[:, :, 0]
    ys = xyz[:, :, 1]
    zs = xyz[:, :, 2]
    out_shapes = (
        jax.ShapeDtypeStruct((npoint, 1, B), jnp.int32),
        jax.ShapeDtypeStruct((npoint, 1, B), jnp.float32),
        jax.ShapeDtypeStruct((npoint, 1, B), jnp.float32),
        jax.ShapeDtypeStruct((npoint, 1, B), jnp.float32),
    )
    idx, cx, cy, cz = pl.pallas_call(
        functools.partial(_fps_body, npoint=npoint),
        out_shape=out_shapes,
    )(xs, ys, zs)
    idx = idx[:, 0, :].T
    new_xyz = jnp.stack([cx[:, 0, :].T, cy[:, 0, :].T, cz[:, 0, :].T], axis=-1)
    return idx, new_xyz


_EPS = 1e-5


def _layer_first_body(x_ref, off_ref, wt_ref, b_ref, p_ref, sum_ref, sq_ref):
    X = x_ref[...] - off_ref[...]
    P = jax.lax.dot_general(
        X.astype(jnp.bfloat16), wt_ref[...].astype(jnp.bfloat16),
        (((1,), (0,)), ((), ())), preferred_element_type=jnp.float32)
    P = P + b_ref[...]
    p_ref[...] = P
    s = jnp.sum(P, axis=0, keepdims=True)
    q = jnp.sum(P * P, axis=0, keepdims=True)

    @pl.when(pl.program_id(0) == 0)
    def _():
        sum_ref[...] = s
        sq_ref[...] = q

    @pl.when(pl.program_id(0) > 0)
    def _():
        sum_ref[...] += s
        sq_ref[...] += q


def _norm_relu(X, s_ref, q_ref, g_ref, be_ref, R):
    mu = s_ref[...] / R
    var = q_ref[...] / R - mu * mu
    X = (X - mu) / jnp.sqrt(var + _EPS) * g_ref[...] + be_ref[...]
    return jnp.maximum(X, 0.0)


def _layer_mid_body(x_ref, s_ref, q_ref, g_ref, be_ref, wt_ref, b_ref,
                    p_ref, sum_ref, sq_ref, *, R):
    X = _norm_relu(x_ref[...], s_ref, q_ref, g_ref, be_ref, R)
    P = jax.lax.dot_general(
        X.astype(jnp.bfloat16), wt_ref[...].astype(jnp.bfloat16),
        (((1,), (0,)), ((), ())), preferred_element_type=jnp.float32)
    P = P + b_ref[...]
    p_ref[...] = P
    s = jnp.sum(P, axis=0, keepdims=True)
    q = jnp.sum(P * P, axis=0, keepdims=True)

    @pl.when(pl.program_id(0) == 0)
    def _():
        sum_ref[...] = s
        sq_ref[...] = q

    @pl.when(pl.program_id(0) > 0)
    def _():
        sum_ref[...] += s
        sq_ref[...] += q


def _maxpool_body(x_ref, s_ref, q_ref, g_ref, be_ref, o_ref, *, R, K, C):
    Y = _norm_relu(x_ref[...], s_ref, q_ref, g_ref, be_ref, R)
    acc = Y[:, 0:C]
    for k in range(1, K):
        acc = jnp.maximum(acc, Y[:, k * C:(k + 1) * C])
    o_ref[...] = acc


def _run_layer(X, off, stats, WT, b, tile):
    R, Cin = X.shape
    Cout = WT.shape[1]
    n = R // tile
    outs = (
        jax.ShapeDtypeStruct((R, Cout), jnp.float32),
        jax.ShapeDtypeStruct((1, Cout), jnp.float32),
        jax.ShapeDtypeStruct((1, Cout), jnp.float32),
    )
    row_spec = pl.BlockSpec((tile, Cin), lambda i: (i, 0))
    full = lambda shp: pl.BlockSpec(shp, lambda i: (0, 0))
    out_specs = (
        pl.BlockSpec((tile, Cout), lambda i: (i, 0)),
        full((1, Cout)),
        full((1, Cout)),
    )
    if off is not None:
        return pl.pallas_call(
            _layer_first_body, grid=(n,),
            in_specs=[row_spec, row_spec, full((Cin, Cout)), full((1, Cout))],
            out_specs=out_specs, out_shape=outs,
        )(X, off, WT, b)
    s, q, g, be = stats
    return pl.pallas_call(
        functools.partial(_layer_mid_body, R=float(R)), grid=(n,),
        in_specs=[row_spec, full((1, Cin)), full((1, Cin)), full((1, Cin)),
                  full((1, Cin)), full((Cin, Cout)), full((1, Cout))],
        out_specs=out_specs, out_shape=outs,
    )(X, s, q, g, be, WT, b)


def _run_maxpool(P, stats, B, S, K, C, tile):
    s, q, g, be = stats
    X = P.reshape(B * S, K * C)
    gt = jnp.tile(g, (1, K))
    bt = jnp.tile(be, (1, K))
    st = jnp.tile(s, (1, K))
    qt = jnp.tile(q, (1, K))
    n = (B * S) // tile
    full = lambda shp: pl.BlockSpec(shp, lambda i: (0, 0))
    return pl.pallas_call(
        functools.partial(_maxpool_body, R=float(B * S * K), K=K, C=C),
        grid=(n,),
        in_specs=[pl.BlockSpec((tile, K * C), lambda i: (i, 0)),
                  full((1, K * C)), full((1, K * C)), full((1, K * C)),
                  full((1, K * C))],
        out_specs=pl.BlockSpec((tile, C), lambda i: (i, 0)),
        out_shape=jax.ShapeDtypeStruct((B * S, C), jnp.float32),
    )(X, st, qt, gt, bt)


def _knn_body(q_ref, bx_ref, by_ref, bz_ref, idx_ref, *, K):
    q = q_ref[0]
    S = q.shape[0]
    N = bx_ref.shape[-1]
    qx, qy, qz = q[:, 0:1], q[:, 1:2], q[:, 2:3]
    bx, by, bz = bx_ref[0], by_ref[0], bz_ref[0]
    sq = qx * qx + qy * qy + qz * qz
    sb = bx * bx + by * by + bz * bz
    def r(v):
        return v.astype(jnp.bfloat16).astype(jnp.float32)
    qb = r(qx) * r(bx) + r(qy) * r(by) + r(qz) * r(bz)
    D = sq + sb - 2.0 * qb
    iota = jax.lax.broadcasted_iota(jnp.int32, (S, N), 1)
    kiota = jax.lax.broadcasted_iota(jnp.int32, (S, K), 1)
    acc = jnp.zeros((S, K), jnp.int32)
    INF = jnp.float32(3.0e38)
    for k in range(K):
        m = jnp.min(D, axis=1, keepdims=True)
        cand = jnp.where(D == m, iota, jnp.int32(N))
        j = jnp.min(cand, axis=1, keepdims=True)
        acc = jnp.where(kiota == k, j, acc)
        D = jnp.where(iota == j, INF, D)
    idx_ref[0] = acc


def _knn(new_xyz, bx, by, bz, k):
    B, S, _ = new_xyz.shape
    N = bx.shape[1]
    bx3 = bx[:, None, :]
    by3 = by[:, None, :]
    bz3 = bz[:, None, :]
    return pl.pallas_call(
        functools.partial(_knn_body, K=k),
        grid=(B,),
        in_specs=[
            pl.BlockSpec((1, S, 3), lambda b: (b, 0, 0)),
            pl.BlockSpec((1, 1, N), lambda b: (b, 0, 0)),
            pl.BlockSpec((1, 1, N), lambda b: (b, 0, 0)),
            pl.BlockSpec((1, 1, N), lambda b: (b, 0, 0)),
        ],
        out_specs=pl.BlockSpec((1, S, k), lambda b: (b, 0, 0)),
        out_shape=jax.ShapeDtypeStruct((B, S, k), jnp.int32),
    )(new_xyz, bx3, by3, bz3)


_SC_SUB = 128


def _sc_gather(table, idx_flat):
    R = idx_flat.shape[0]
    Cp = table.shape[1]
    NW = 32
    per_w = R // NW
    n_sub = per_w // _SC_SUB
    assert per_w % _SC_SUB == 0

    mesh = plsc.VectorSubcoreMesh(core_axis_name="c", subcore_axis_name="s")

    @functools.partial(
        pl.kernel, mesh=mesh,
        compiler_params=pltpu.CompilerParams(use_tc_tiling_on_sc=False),
        out_type=jax.ShapeDtypeStruct((R, Cp), jnp.float32),
        scratch_types=[
            pltpu.VMEM((_SC_SUB,), jnp.int32),
            pltpu.VMEM((_SC_SUB, Cp), jnp.float32),
            pltpu.SemaphoreType.DMA,
        ],
    )
    def k(table_hbm, idx_hbm, out_hbm, idx_v, rows_v, sem):
        wid = lax.axis_index("s") * 2 + lax.axis_index("c")

        def body(i, carry):
            base = wid * per_w + i * _SC_SUB
            pltpu.sync_copy(idx_hbm.at[pl.ds(base, _SC_SUB)], idx_v)
            pltpu.async_copy(table_hbm.at[idx_v], rows_v, sem).wait()
            pltpu.sync_copy(rows_v, out_hbm.at[pl.ds(base, _SC_SUB)])
            return carry

        lax.fori_loop(0, n_sub, body, 0)

    return k(table, idx_flat)


def _sa_stage(xyz, points, npoint, K, layers):
    B, N, _ = xyz.shape
    S = npoint
    (W1, b1, g1, be1), (W2, b2, g2, be2), (W3, b3, g3, be3) = layers
    _, new_xyz = _fps(xyz, npoint)
    idx = _knn(new_xyz, xyz[:, :, 0], xyz[:, :, 1], xyz[:, :, 2], K)
    Cin = 3 + (points.shape[-1] if points is not None else 0)
    Cp = ((Cin + 15) // 16) * 16
    T = xyz if points is None else jnp.concatenate([xyz, points], -1)
    Tp = jnp.pad(T.reshape(B * N, Cin), ((0, 0), (0, Cp - Cin)))
    idx_flat = (idx + (jnp.arange(B, dtype=jnp.int32) * N)[:, None, None]
                ).reshape(-1)
    G = _sc_gather(Tp, idx_flat)
    R = B * S * K
    off = jnp.pad(jnp.repeat(new_xyz, K, axis=1).reshape(R, 3),
                  ((0, 0), (0, Cp - 3)))
    W1Tp = jnp.pad(W1.T, ((0, Cp - Cin), (0, 0)))
    tile = min(R, 8192)
    P1, s1, q1 = _run_layer(G, off, None, W1Tp, b1[None], tile)
    P2, s2, q2 = _run_layer(P1, None, (s1, q1, g1[None], be1[None]),
                            W2.T, b2[None], tile)
    P3, s3, q3 = _run_layer(P2, None, (s2, q2, g2[None], be2[None]),
                            W3.T, b3[None], tile)
    C3 = W3.shape[0]
    tile_d = min(B * S, max(8, (8 * 1024 * 1024) // (K * C3 * 4)))
    out = _run_maxpool(P3, (s3, q3, g3[None], be3[None]), B, S, K, C3, tile_d)
    return new_xyz, out.reshape(B, S, C3)


def _sa4(xyz, points, layers):
    B, S, _ = xyz.shape
    X = jnp.concatenate([xyz, points], -1).reshape(B * S, -1)
    (W1, b1, g1, be1), (W2, b2, g2, be2), (W3, b3, g3, be3) = layers
    R = B * S
    P1, s1, q1 = _run_layer(X, jnp.zeros_like(X), None, W1.T, b1[None], R)
    P2, s2, q2 = _run_layer(P1, None, (s1, q1, g1[None], be1[None]),
                            W2.T, b2[None], R)
    P3, s3, q3 = _run_layer(P2, None, (s2, q2, g2[None], be2[None]),
                            W3.T, b3[None], R)
    return _run_maxpool(P3, (s3, q3, g3[None], be3[None]), B, 1, S, 1024, B)


def kernel(x, params):
    xyz, points = x, None
    xyz, points = _sa_stage(xyz, points, 512, 32, params['sa1'])
    xyz, points = _sa_stage(xyz, points, 128, 32, params['sa2'])
    xyz, points = _sa_stage(xyz, points, 32, 32, params['sa3'])
    return _sa4(xyz, points, params['sa4'])
